# baseline (speedup 1.0000x reference)
import functools

import jax
import jax.numpy as jnp
from jax import lax
from jax.experimental import pallas as pl
from jax.experimental.pallas import tpu as pltpu
from jax.experimental.pallas import tpu_sc as plsc

NC, NS = 2, 16
NW = NC * NS
CHUNK = 32
LANES = 16


def _sc_gather_sum(row_idx, col_idx, row_table, col_table):
  n = row_idx.shape[0]
  d = row_table.shape[1]
  per_w = n // NW
  steps = per_w // CHUNK
  mesh = plsc.VectorSubcoreMesh(core_axis_name="c", subcore_axis_name="s")

  @functools.partial(
      pl.kernel,
      out_type=jax.ShapeDtypeStruct((n, d), row_table.dtype),
      mesh=mesh,
      scratch_types=[
          pltpu.VMEM((per_w,), jnp.int32),
          pltpu.VMEM((per_w,), jnp.int32),
          pltpu.VMEM((CHUNK, d), row_table.dtype),
          pltpu.VMEM((CHUNK, d), row_table.dtype),
          pltpu.VMEM((CHUNK, d), row_table.dtype),
          pltpu.VMEM((CHUNK, d), row_table.dtype),
          pltpu.SemaphoreType.DMA,
          pltpu.SemaphoreType.DMA,
          pltpu.SemaphoreType.DMA,
          pltpu.SemaphoreType.DMA,
      ],
  )
  def k(ri_hbm, ci_hbm, rt_hbm, ct_hbm, g_hbm,
        ir_v, ic_v, ra, ca, rb, cb, sra, sca, srb, scb):
    wid = lax.axis_index("s") * NC + lax.axis_index("c")
    base = wid * per_w
    pltpu.sync_copy(ri_hbm.at[pl.ds(base, per_w)], ir_v)
    pltpu.sync_copy(ci_hbm.at[pl.ds(base, per_w)], ic_v)
    sets = ((ra, ca, sra, sca), (rb, cb, srb, scb))
    handles = [None] * (2 * steps)

    def start(k_):
      rbuf, cbuf, rs, cs = sets[k_ % 2]
      handles[2 * k_] = pltpu.async_copy(
          rt_hbm.at[ir_v.at[pl.ds(k_ * CHUNK, CHUNK)]], rbuf, rs)
      handles[2 * k_ + 1] = pltpu.async_copy(
          ct_hbm.at[ic_v.at[pl.ds(k_ * CHUNK, CHUNK)]], cbuf, cs)

    def finish(k_):
      rbuf, cbuf, _, _ = sets[k_ % 2]
      handles[2 * k_].wait()
      handles[2 * k_ + 1].wait()

      @pl.loop(0, CHUNK)
      def _(r):
        for c in range(d // LANES):
          sl = (r, pl.ds(c * LANES, LANES))
          rbuf[sl] = rbuf[sl] + cbuf[sl]

      pltpu.sync_copy(rbuf, g_hbm.at[pl.ds(base + k_ * CHUNK, CHUNK)])

    start(0)
    for k_ in range(1, steps):
      start(k_)
      finish(k_ - 1)
    finish(steps - 1)

  return k(row_idx, col_idx, row_table, col_table)


def _tc_body(fpt_ref, w_ref, b_ref, g_ref, alias_ref, out_ref):
  del alias_ref
  w = w_ref[...].astype(jnp.bfloat16)
  x = fpt_ref[...].astype(jnp.bfloat16)
  acc = lax.dot_general(x, w, (((0,), (0,)), ((), ())),
                        preferred_element_type=jnp.float32)
  out_ref[...] = acc + b_ref[...] + g_ref[...]


def _tc_project_add_half(fpt3, w_pad, b2, g3, acc_buf, half, block_cols=256):
  pw, bsz, s = fpt3.shape
  h = w_pad.shape[1]
  sh = g3.shape[1]
  grid = (sh // block_cols,)
  off = half * (sh // block_cols)
  return pl.pallas_call(
      _tc_body,
      grid=grid,
      in_specs=[
          pl.BlockSpec((pw, bsz, block_cols), lambda i: (0, 0, i + off)),
          pl.BlockSpec((pw, h), lambda i: (0, 0)),
          pl.BlockSpec((1, h), lambda i: (0, 0)),
          pl.BlockSpec((bsz, block_cols, h), lambda i: (0, i, 0)),
          pl.BlockSpec((1, 8, 128), lambda i: (0, 0, 0)),
      ],
      out_specs=pl.BlockSpec((bsz, block_cols, h), lambda i: (0, i + off, 0)),
      out_shape=jax.ShapeDtypeStruct((bsz, s, h), jnp.float32),
      input_output_aliases={4: 0},
  )(fpt3, w_pad, b2, g3, acc_buf)


def kernel(flattened_patches, W, b, row_table, col_table):
  bsz, s, pw = flattened_patches.shape
  h = W.shape[1]
  sh = s // 2
  fpt3 = flattened_patches.transpose(2, 0, 1)
  ri2 = fpt3[0].astype(jnp.int32)
  ci2 = fpt3[1].astype(jnp.int32)
  w_pad = jnp.concatenate([jnp.zeros((2, h), W.dtype), W], axis=0)
  b2 = b.reshape(1, h)
  buf = jnp.zeros((bsz, s, h), jnp.float32)
  for half in range(2):
    ri = ri2[:, half * sh:(half + 1) * sh].reshape(bsz * sh)
    ci = ci2[:, half * sh:(half + 1) * sh].reshape(bsz * sh)
    g = _sc_gather_sum(ri, ci, row_table, col_table)
    buf = _tc_project_add_half(fpt3, w_pad, b2, g.reshape(bsz, sh, h),
                               buf, half)
  return buf
